# baseline (speedup 1.0000x reference)
import functools

import jax
import jax.numpy as jnp
from jax import lax
from jax.experimental import pallas as pl
from jax.experimental.pallas import tpu as pltpu
from jax.experimental.pallas import tpu_sc as plsc

N = 10000
NP = 10112
HALF = NP // 2
NL = 5120
D = 128
E = 320000
NC = 2
NS = 16
NW = NC * NS
EPW = E // NW
K = 128
PADE = 240
NCH = (EPW + PADE) // K
CAP = 88
GRP = NCH * (K // 16)
RIN = NL // NS
RLST = HALF - 15 * RIN

_mesh = plsc.VectorSubcoreMesh(
    core_axis_name="c", subcore_axis_name="s", num_cores=NC, num_subcores=NS)


@functools.partial(
    pl.kernel,
    out_type=(
        jax.ShapeDtypeStruct((NW * NP,), jnp.float32),
        jax.ShapeDtypeStruct((NW * 2 * CAP * K,), jnp.int32),
        jax.ShapeDtypeStruct((NW * 2 * CAP * K,), jnp.int32),
    ),
    mesh=_mesh,
    scratch_types=[
        pltpu.VMEM((NCH, K), jnp.int32),
        pltpu.VMEM((NCH, K), jnp.int32),
        pltpu.VMEM((CAP * K,), jnp.int32),
        pltpu.VMEM((CAP * K,), jnp.int32),
        pltpu.VMEM((CAP * K,), jnp.int32),
        pltpu.VMEM((CAP * K,), jnp.int32),
        pltpu.VMEM((NP,), jnp.float32),
    ],
    compiler_params=pltpu.CompilerParams(needs_layout_passes=False),
)
def _part_kernel(src_hbm, dst_hbm, deg_hbm, srcb_hbm, dstb_hbm,
                 src_v, dst_v, sb0, sb1, db0, db1, hist_v):
    c = lax.axis_index("c")
    s = lax.axis_index("s")
    wid = s * NC + c
    pltpu.sync_copy(src_hbm.at[wid], src_v)
    pltpu.sync_copy(dst_hbm.at[wid], dst_v)

    lanes = lax.iota(jnp.int32, 16)
    zero16f = jnp.zeros((16,), jnp.float32)
    ones16f = jnp.ones((16,), jnp.float32)
    dum_src = N + lanes
    dum_dst = HALF + lanes

    def fbody(i, carry):
        sb0[pl.ds(i * 16, 16)] = dum_src
        sb1[pl.ds(i * 16, 16)] = dum_src
        db0[pl.ds(i * 16, 16)] = dum_dst
        db1[pl.ds(i * 16, 16)] = dum_dst
        return carry

    lax.fori_loop(0, CAP * K // 16, fbody, 0)

    def zbody(i, carry):
        hist_v[pl.ds(i * 16, 16)] = zero16f
        return carry

    lax.fori_loop(0, NP // 16, zbody, 0)

    onehot0 = jnp.where(lanes == 0, 1, 0)

    def body(i, carry):
        cur0, cur1 = carry
        r = i // (K // 16)
        l = i % (K // 16)
        sv = src_v[r, pl.ds(l * 16, 16)]
        dv = dst_v[r, pl.ds(l * 16, 16)]
        plsc.addupdate_scatter(hist_v, [dv], ones16f)
        m0 = dv < HALF
        m1 = jnp.logical_not(m0)
        plsc.store_compressed(sb0.at[pl.ds(cur0, 16)], sv, mask=m0)
        plsc.store_compressed(db0.at[pl.ds(cur0, 16)], dv, mask=m0)
        plsc.store_compressed(sb1.at[pl.ds(cur1, 16)], sv, mask=m1)
        plsc.store_compressed(db1.at[pl.ds(cur1, 16)], dv - HALF, mask=m1)
        n0 = lax.reduce_max(plsc.all_reduce_population_count(m0) * onehot0,
                            (0,))
        return (cur0 + n0, cur1 + (16 - n0))

    lax.fori_loop(0, GRP, body, (0, 0))

    pltpu.sync_copy(hist_v, deg_hbm.at[pl.ds(wid * NP, NP)])
    base0 = wid * (CAP * K)
    base1 = (NW + wid) * (CAP * K)
    pltpu.sync_copy(sb0, srcb_hbm.at[pl.ds(base0, CAP * K)])
    pltpu.sync_copy(sb1, srcb_hbm.at[pl.ds(base1, CAP * K)])
    pltpu.sync_copy(db0, dstb_hbm.at[pl.ds(base0, CAP * K)])
    pltpu.sync_copy(db1, dstb_hbm.at[pl.ds(base1, CAP * K)])


@functools.partial(
    pl.kernel,
    out_type=jax.ShapeDtypeStruct((NP, D), jnp.float32),
    mesh=_mesh,
    scratch_types=[
        pltpu.VMEM((2, CAP, K), jnp.int32),
        pltpu.VMEM((2, CAP, K), jnp.int32),
        pltpu.VMEM((K, D), jnp.float32),
        pltpu.VMEM((K, D), jnp.float32),
        pltpu.VMEM_SHARED((NL, D), jnp.float32),
        pltpu.SemaphoreType.DMA,
        pltpu.SemaphoreType.DMA,
    ],
    compiler_params=pltpu.CompilerParams(needs_layout_passes=False),
)
def _seg_kernel(y_hbm, srcb_hbm, dstb_hbm, out_hbm,
                src_v, dst_v, st_a, st_b, acc_sh, gs_a, gs_b):
    c = lax.axis_index("c")
    s = lax.axis_index("s")
    pltpu.sync_copy(srcb_hbm.at[c, pl.ds(2 * s, 2)], src_v)
    pltpu.sync_copy(dstb_hbm.at[c, pl.ds(2 * s, 2)], dst_v)
    gbase = c * HALF + s * RIN

    @pl.when(s < NS - 1)
    def _():
        pltpu.sync_copy(y_hbm.at[pl.ds(gbase, RIN)],
                        acc_sh.at[pl.ds(s * RIN, RIN)])

    @pl.when(s == NS - 1)
    def _():
        pltpu.sync_copy(y_hbm.at[pl.ds(gbase, RLST)],
                        acc_sh.at[pl.ds(s * RIN, RLST)])

    plsc.subcore_barrier()

    lanes = lax.iota(jnp.int32, 16)
    onehot0 = jnp.where(lanes == 0, 1, 0)

    def first_lane(o, j):
        return lax.reduce_max(dst_v[o, j, pl.ds(0, 16)] * onehot0, (0,))

    for o in range(2):
        pltpu.async_copy(y_hbm.at[src_v.at[o, 0]], st_a, gs_a)

        def cond(carry):
            j, go = carry
            return go

        def wbody(carry):
            j, go = carry
            j1 = jnp.minimum(j + 1, CAP - 1)
            j2 = jnp.minimum(j + 2, CAP - 1)
            pltpu.async_copy(y_hbm.at[src_v.at[o, j1]], st_b, gs_b)
            pltpu.make_async_copy(y_hbm.at[src_v.at[o, 0]], st_a, gs_a).wait()
            real_a = first_lane(o, j) < HALF

            @pl.when(real_a)
            def _():
                pltpu.sync_copy(st_a, acc_sh.at[dst_v.at[o, j]], add=True)

            pltpu.async_copy(y_hbm.at[src_v.at[o, j2]], st_a, gs_a)
            pltpu.make_async_copy(y_hbm.at[src_v.at[o, 0]], st_b, gs_b).wait()
            real_b = jnp.logical_and(real_a, first_lane(o, j1) < HALF)

            @pl.when(real_b)
            def _():
                pltpu.sync_copy(st_b, acc_sh.at[dst_v.at[o, j1]], add=True)

            return (j + 2, jnp.logical_and(real_b, j + 2 < CAP))

        lax.while_loop(cond, wbody, (0, jnp.bool_(True)))
        pltpu.make_async_copy(y_hbm.at[src_v.at[o, 0]], st_a, gs_a).wait()

    plsc.subcore_barrier()

    @pl.when(s < NS - 1)
    def _():
        pltpu.sync_copy(acc_sh.at[pl.ds(s * RIN, RIN)],
                        out_hbm.at[pl.ds(gbase, RIN)])

    @pl.when(s == NS - 1)
    def _():
        pltpu.sync_copy(acc_sh.at[pl.ds(s * RIN, RLST)],
                        out_hbm.at[pl.ds(gbase, RLST)])


def _tca_body(x_ref, w_ref, degp_ref, y_ref, dinv_ref):
    deg = 1.0 + jnp.sum(jnp.transpose(degp_ref[...]), axis=1, keepdims=True)
    dinv = lax.rsqrt(deg)
    y_ref[...] = jnp.dot(x_ref[...], w_ref[...],
                         preferred_element_type=jnp.float32) * dinv
    dinv_ref[...] = dinv


_tca = pl.pallas_call(
    _tca_body,
    out_shape=(jax.ShapeDtypeStruct((NP, D), jnp.float32),
               jax.ShapeDtypeStruct((NP, 1), jnp.float32)),
)


def _tcb_body(acc_ref, dinv_ref, b_ref, w_ref, o_ref):
    dinv = dinv_ref[...]
    h = acc_ref[...] * dinv + b_ref[...]
    o_ref[...] = jnp.dot(h, w_ref[...],
                         preferred_element_type=jnp.float32) * dinv


_tcb = pl.pallas_call(
    _tcb_body,
    out_shape=jax.ShapeDtypeStruct((NP, D), jnp.float32),
)


def _tcc_body(acc_ref, dinv_ref, b_ref, o_ref):
    o_ref[...] = acc_ref[:N] * dinv_ref[:N] + b_ref[...]


_tcc = pl.pallas_call(
    _tcc_body,
    out_shape=jax.ShapeDtypeStruct((N, D), jnp.float32),
)


def kernel(features, edge_index, W0, b0, W1, b1, W2, b2):
    dummy = (N + (jnp.arange(PADE, dtype=jnp.int32) % (NP - N)))
    dummy = jnp.broadcast_to(dummy, (NW, PADE))
    src = jnp.concatenate(
        [edge_index[0].reshape(NW, EPW), dummy], axis=1).reshape(NW, NCH, K)
    dst = jnp.concatenate(
        [edge_index[1].reshape(NW, EPW), dummy], axis=1).reshape(NW, NCH, K)
    xpad = jnp.concatenate(
        [features, jnp.zeros((NP - N, D), jnp.float32)], axis=0)

    deg1d, srcb1d, dstb1d = _part_kernel(src, dst)
    degp = deg1d.reshape(NW, NP)
    srcb = srcb1d.reshape(2, NW, CAP, K)
    dstb = dstb1d.reshape(2, NW, CAP, K)

    y0, dinv = _tca(xpad, W0, degp)
    a0 = _seg_kernel(y0, srcb, dstb)
    y1 = _tcb(a0, dinv, b0.reshape(1, D), W1)
    a1 = _seg_kernel(y1, srcb, dstb)
    y2 = _tcb(a1, dinv, b1.reshape(1, D), W2)
    a2 = _seg_kernel(y2, srcb, dstb)
    return _tcc(a2, dinv, b2.reshape(1, D))


# speedup vs baseline: 1.1099x; 1.1099x over previous
import functools

import jax
import jax.numpy as jnp
from jax import lax
from jax.experimental import pallas as pl
from jax.experimental.pallas import tpu as pltpu
from jax.experimental.pallas import tpu_sc as plsc

N = 10000
NP = 10112
HALF = NP // 2
NL = 5120
D = 128
E = 320000
NC = 2
NS = 16
NW = NC * NS
EPW = E // NW
K = 128
PADE = 240
NCH = (EPW + PADE) // K
CAP = 88
GRP = NCH * (K // 16)
RIN = NL // NS
RLST = HALF - 15 * RIN

_mesh = plsc.VectorSubcoreMesh(
    core_axis_name="c", subcore_axis_name="s", num_cores=NC, num_subcores=NS)


@functools.partial(
    pl.kernel,
    out_type=(
        jax.ShapeDtypeStruct((NW * NP,), jnp.float32),
        jax.ShapeDtypeStruct((NW * 2 * CAP * K,), jnp.int32),
        jax.ShapeDtypeStruct((NW * 2 * CAP * K,), jnp.int32),
    ),
    mesh=_mesh,
    scratch_types=[
        pltpu.VMEM((NCH, K), jnp.int32),
        pltpu.VMEM((NCH, K), jnp.int32),
        pltpu.VMEM((CAP * K,), jnp.int32),
        pltpu.VMEM((CAP * K,), jnp.int32),
        pltpu.VMEM((CAP * K,), jnp.int32),
        pltpu.VMEM((CAP * K,), jnp.int32),
        pltpu.VMEM((NP,), jnp.float32),
    ],
    compiler_params=pltpu.CompilerParams(needs_layout_passes=False),
)
def _part_kernel(src_hbm, dst_hbm, deg_hbm, srcb_hbm, dstb_hbm,
                 src_v, dst_v, sb0, sb1, db0, db1, hist_v):
    c = lax.axis_index("c")
    s = lax.axis_index("s")
    wid = s * NC + c
    pltpu.sync_copy(src_hbm.at[wid], src_v)
    pltpu.sync_copy(dst_hbm.at[wid], dst_v)

    lanes = lax.iota(jnp.int32, 16)
    zero16f = jnp.zeros((16,), jnp.float32)
    ones16f = jnp.ones((16,), jnp.float32)
    dum_src = N + lanes
    dum_dst = HALF + lanes

    def fbody(i, carry):
        sb0[pl.ds(i * 16, 16)] = dum_src
        sb1[pl.ds(i * 16, 16)] = dum_src
        db0[pl.ds(i * 16, 16)] = dum_dst
        db1[pl.ds(i * 16, 16)] = dum_dst
        return carry

    lax.fori_loop(0, CAP * K // 16, fbody, 0)

    def zbody(i, carry):
        hist_v[pl.ds(i * 16, 16)] = zero16f
        return carry

    lax.fori_loop(0, NP // 16, zbody, 0)

    onehot0 = jnp.where(lanes == 0, 1, 0)

    def body(i, carry):
        cur0, cur1 = carry
        r = i // (K // 16)
        l = i % (K // 16)
        sv = src_v[r, pl.ds(l * 16, 16)]
        dv = dst_v[r, pl.ds(l * 16, 16)]
        plsc.addupdate_scatter(hist_v, [dv], ones16f)
        m0 = dv < HALF
        m1 = jnp.logical_not(m0)
        plsc.store_compressed(sb0.at[pl.ds(cur0, 16)], sv, mask=m0)
        plsc.store_compressed(db0.at[pl.ds(cur0, 16)], dv, mask=m0)
        plsc.store_compressed(sb1.at[pl.ds(cur1, 16)], sv, mask=m1)
        plsc.store_compressed(db1.at[pl.ds(cur1, 16)], dv - HALF, mask=m1)
        n0 = lax.reduce_max(plsc.all_reduce_population_count(m0) * onehot0,
                            (0,))
        return (cur0 + n0, cur1 + (16 - n0))

    cur0, cur1 = lax.fori_loop(0, GRP, body, (0, 0))
    zero16i = jnp.zeros((16,), jnp.int32)
    db0[pl.ds((CAP - 1) * K, 16)] = zero16i + (cur0 + K - 1) // K
    db1[pl.ds((CAP - 1) * K, 16)] = zero16i + (cur1 + K - 1) // K

    pltpu.sync_copy(hist_v, deg_hbm.at[pl.ds(wid * NP, NP)])
    base0 = wid * (CAP * K)
    base1 = (NW + wid) * (CAP * K)
    pltpu.sync_copy(sb0, srcb_hbm.at[pl.ds(base0, CAP * K)])
    pltpu.sync_copy(sb1, srcb_hbm.at[pl.ds(base1, CAP * K)])
    pltpu.sync_copy(db0, dstb_hbm.at[pl.ds(base0, CAP * K)])
    pltpu.sync_copy(db1, dstb_hbm.at[pl.ds(base1, CAP * K)])


@functools.partial(
    pl.kernel,
    out_type=jax.ShapeDtypeStruct((NP, D), jnp.float32),
    mesh=_mesh,
    scratch_types=[
        pltpu.VMEM((2, CAP, K), jnp.int32),
        pltpu.VMEM((2, CAP, K), jnp.int32),
        pltpu.VMEM((K, D), jnp.float32),
        pltpu.VMEM((K, D), jnp.float32),
        pltpu.VMEM_SHARED((NL, D), jnp.float32),
        pltpu.SemaphoreType.DMA,
        pltpu.SemaphoreType.DMA,
    ],
    compiler_params=pltpu.CompilerParams(needs_layout_passes=False),
)
def _seg_kernel(y_hbm, srcb_hbm, dstb_hbm, out_hbm,
                src_v, dst_v, st_a, st_b, acc_sh, gs_a, gs_b):
    c = lax.axis_index("c")
    s = lax.axis_index("s")
    pltpu.sync_copy(srcb_hbm.at[c, pl.ds(2 * s, 2)], src_v)
    pltpu.sync_copy(dstb_hbm.at[c, pl.ds(2 * s, 2)], dst_v)
    gbase = c * HALF + s * RIN

    @pl.when(s < NS - 1)
    def _():
        pltpu.sync_copy(y_hbm.at[pl.ds(gbase, RIN)],
                        acc_sh.at[pl.ds(s * RIN, RIN)])

    @pl.when(s == NS - 1)
    def _():
        pltpu.sync_copy(y_hbm.at[pl.ds(gbase, RLST)],
                        acc_sh.at[pl.ds(s * RIN, RLST)])

    plsc.subcore_barrier()

    lanes = lax.iota(jnp.int32, 16)
    onehot0 = jnp.where(lanes == 0, 1, 0)

    for o in range(2):
        nch = lax.reduce_max(
            dst_v[o, CAP - 1, pl.ds(0, 16)] * onehot0, (0,))
        pltpu.async_copy(y_hbm.at[src_v.at[o, 0]], st_a, gs_a)

        def body(i, carry):
            j = 2 * i
            j1 = jnp.minimum(j + 1, CAP - 2)
            j2 = jnp.minimum(j + 2, CAP - 2)
            pltpu.async_copy(y_hbm.at[src_v.at[o, j1]], st_b, gs_b)
            pltpu.make_async_copy(y_hbm.at[src_v.at[o, 0]], st_a, gs_a).wait()
            pltpu.sync_copy(st_a, acc_sh.at[dst_v.at[o, j]], add=True)
            pltpu.async_copy(y_hbm.at[src_v.at[o, j2]], st_a, gs_a)
            pltpu.make_async_copy(y_hbm.at[src_v.at[o, 0]], st_b, gs_b).wait()
            pltpu.sync_copy(st_b, acc_sh.at[dst_v.at[o, j1]], add=True)
            return carry

        lax.fori_loop(0, (nch + 1) // 2, body, 0)
        pltpu.make_async_copy(y_hbm.at[src_v.at[o, 0]], st_a, gs_a).wait()

    plsc.subcore_barrier()

    @pl.when(s < NS - 1)
    def _():
        pltpu.sync_copy(acc_sh.at[pl.ds(s * RIN, RIN)],
                        out_hbm.at[pl.ds(gbase, RIN)])

    @pl.when(s == NS - 1)
    def _():
        pltpu.sync_copy(acc_sh.at[pl.ds(s * RIN, RLST)],
                        out_hbm.at[pl.ds(gbase, RLST)])


def _tca_body(x_ref, w_ref, degp_ref, y_ref, dinv_ref):
    deg = 1.0 + jnp.sum(jnp.transpose(degp_ref[...]), axis=1, keepdims=True)
    dinv = lax.rsqrt(deg)
    y_ref[...] = jnp.dot(x_ref[...], w_ref[...],
                         preferred_element_type=jnp.float32) * dinv
    dinv_ref[...] = dinv


_tca = pl.pallas_call(
    _tca_body,
    out_shape=(jax.ShapeDtypeStruct((NP, D), jnp.float32),
               jax.ShapeDtypeStruct((NP, 1), jnp.float32)),
)


def _tcb_body(acc_ref, dinv_ref, b_ref, w_ref, o_ref):
    dinv = dinv_ref[...]
    h = acc_ref[...] * dinv + b_ref[...]
    o_ref[...] = jnp.dot(h, w_ref[...],
                         preferred_element_type=jnp.float32) * dinv


_tcb = pl.pallas_call(
    _tcb_body,
    out_shape=jax.ShapeDtypeStruct((NP, D), jnp.float32),
)


def _tcc_body(acc_ref, dinv_ref, b_ref, o_ref):
    o_ref[...] = acc_ref[:N] * dinv_ref[:N] + b_ref[...]


_tcc = pl.pallas_call(
    _tcc_body,
    out_shape=jax.ShapeDtypeStruct((N, D), jnp.float32),
)


def kernel(features, edge_index, W0, b0, W1, b1, W2, b2):
    dummy = (N + (jnp.arange(PADE, dtype=jnp.int32) % (NP - N)))
    dummy = jnp.broadcast_to(dummy, (NW, PADE))
    src = jnp.concatenate(
        [edge_index[0].reshape(NW, EPW), dummy], axis=1).reshape(NW, NCH, K)
    dst = jnp.concatenate(
        [edge_index[1].reshape(NW, EPW), dummy], axis=1).reshape(NW, NCH, K)
    xpad = jnp.concatenate(
        [features, jnp.zeros((NP - N, D), jnp.float32)], axis=0)

    deg1d, srcb1d, dstb1d = _part_kernel(src, dst)
    degp = deg1d.reshape(NW, NP)
    srcb = srcb1d.reshape(2, NW, CAP, K)
    dstb = dstb1d.reshape(2, NW, CAP, K)

    y0, dinv = _tca(xpad, W0, degp)
    a0 = _seg_kernel(y0, srcb, dstb)
    y1 = _tcb(a0, dinv, b0.reshape(1, D), W1)
    a1 = _seg_kernel(y1, srcb, dstb)
    y2 = _tcb(a1, dinv, b1.reshape(1, D), W2)
    a2 = _seg_kernel(y2, srcb, dstb)
    return _tcc(a2, dinv, b2.reshape(1, D))


# speedup vs baseline: 1.3262x; 1.1949x over previous
import functools

import jax
import jax.numpy as jnp
from jax import lax
from jax.experimental import pallas as pl
from jax.experimental.pallas import tpu as pltpu
from jax.experimental.pallas import tpu_sc as plsc

N = 10000
NP = 10112
D = 128
E = 320000
NC = 2
NS = 16
NW = NC * NS
EPW = E // NW
K = 128
PADE = 240
NCH = (EPW + PADE) // K
NPH = 2
NCHP = NCH // NPH
RPT = NP // NS
DEGW = 128

_mesh = plsc.VectorSubcoreMesh(
    core_axis_name="c", subcore_axis_name="s", num_cores=NC, num_subcores=NS)


@functools.partial(
    pl.kernel,
    out_type=jax.ShapeDtypeStruct((NW * NP,), jnp.float32),
    mesh=_mesh,
    scratch_types=[
        pltpu.VMEM((NCH, K), jnp.int32),
        pltpu.VMEM((NP,), jnp.float32),
    ],
    compiler_params=pltpu.CompilerParams(needs_layout_passes=False),
)
def _deg_kernel(dst_hbm, out_hbm, dst_v, hist_v):
    c = lax.axis_index("c")
    s = lax.axis_index("s")
    wid = s * NC + c
    pltpu.sync_copy(dst_hbm.at[wid], dst_v)
    zero16 = jnp.zeros((16,), jnp.float32)
    ones16 = jnp.ones((16,), jnp.float32)

    def zbody(i, carry):
        hist_v[pl.ds(i * 16, 16)] = zero16
        return carry

    lax.fori_loop(0, NP // 16, zbody, 0)

    def body(i, carry):
        j = i // (K // 16)
        l = i % (K // 16)
        idx16 = dst_v[j, pl.ds(l * 16, 16)]
        plsc.addupdate_scatter(hist_v, [idx16], ones16)
        return carry

    lax.fori_loop(0, NCH * (K // 16), body, 0)
    pltpu.sync_copy(hist_v, out_hbm.at[pl.ds(wid * NP, NP)])


@functools.partial(
    pl.kernel,
    out_type=jax.ShapeDtypeStruct((NC, NP, D), jnp.float32),
    mesh=_mesh,
    scratch_types=[
        pltpu.VMEM((NCHP, K), jnp.int32),
        pltpu.VMEM((NCHP, K), jnp.int32),
        pltpu.VMEM((K, D), jnp.float32),
        pltpu.VMEM((K, D), jnp.float32),
        pltpu.VMEM_SHARED((NP, D), jnp.float32),
        pltpu.SemaphoreType.DMA,
        pltpu.SemaphoreType.DMA,
    ],
)
def _seg_kernel(y_hbm, src_hbm, dst_hbm, out_hbm,
                src_v, dst_v, st_a, st_b, acc_sh, gs_a, gs_b):
    c = lax.axis_index("c")
    s = lax.axis_index("s")
    wid = s * NC + c
    pltpu.sync_copy(y_hbm.at[pl.ds(s * RPT, RPT)],
                    acc_sh.at[pl.ds(s * RPT, RPT)])
    plsc.subcore_barrier()

    for ph in range(NPH):
        pltpu.sync_copy(src_hbm.at[wid, pl.ds(ph * NCHP, NCHP)], src_v)
        pltpu.sync_copy(dst_hbm.at[wid, pl.ds(ph * NCHP, NCHP)], dst_v)
        pltpu.async_copy(y_hbm.at[src_v.at[0]], st_a, gs_a)

        def body(i, carry):
            j = 2 * i
            pltpu.async_copy(y_hbm.at[src_v.at[j + 1]], st_b, gs_b)
            pltpu.make_async_copy(y_hbm.at[src_v.at[0]], st_a, gs_a).wait()
            pltpu.sync_copy(st_a, acc_sh.at[dst_v.at[j]], add=True)
            jn = jnp.minimum(j + 2, NCHP - 1)
            pltpu.async_copy(y_hbm.at[src_v.at[jn]], st_a, gs_a)
            pltpu.make_async_copy(y_hbm.at[src_v.at[0]], st_b, gs_b).wait()
            pltpu.sync_copy(st_b, acc_sh.at[dst_v.at[j + 1]], add=True)
            return carry

        lax.fori_loop(0, NCHP // 2, body, 0)
        pltpu.make_async_copy(y_hbm.at[src_v.at[0]], st_a, gs_a).wait()
    plsc.subcore_barrier()
    pltpu.sync_copy(acc_sh.at[pl.ds(s * RPT, RPT)],
                    out_hbm.at[c, pl.ds(s * RPT, RPT)])


def _tca_body(x_ref, w_ref, degp_ref, y_ref, dinv_ref):
    deg = 1.0 + jnp.sum(jnp.transpose(degp_ref[...]), axis=1, keepdims=True)
    dinv = lax.rsqrt(deg)
    y_ref[...] = jnp.dot(x_ref[...], w_ref[...],
                         preferred_element_type=jnp.float32) * dinv
    dinv_ref[...] = dinv


_tca = pl.pallas_call(
    _tca_body,
    out_shape=(jax.ShapeDtypeStruct((NP, D), jnp.float32),
               jax.ShapeDtypeStruct((NP, 1), jnp.float32)),
)


def _tcb_body(y_ref, p_ref, dinv_ref, b_ref, w_ref, o_ref):
    dinv = dinv_ref[...]
    h = (p_ref[0] + p_ref[1] - y_ref[...]) * dinv + b_ref[...]
    o_ref[...] = jnp.dot(h, w_ref[...],
                         preferred_element_type=jnp.float32) * dinv


_tcb = pl.pallas_call(
    _tcb_body,
    out_shape=jax.ShapeDtypeStruct((NP, D), jnp.float32),
)


def _tcc_body(y_ref, p_ref, dinv_ref, b_ref, o_ref):
    o_ref[...] = ((p_ref[0, :N] + p_ref[1, :N] - y_ref[:N])
                  * dinv_ref[:N] + b_ref[...])


_tcc = pl.pallas_call(
    _tcc_body,
    out_shape=jax.ShapeDtypeStruct((N, D), jnp.float32),
)


def kernel(features, edge_index, W0, b0, W1, b1, W2, b2):
    dummy = (N + (jnp.arange(PADE, dtype=jnp.int32) % (NP - N)))
    dummy = jnp.broadcast_to(dummy, (NW, PADE))
    src = jnp.concatenate(
        [edge_index[0].reshape(NW, EPW), dummy], axis=1).reshape(NW, NCH, K)
    dst = jnp.concatenate(
        [edge_index[1].reshape(NW, EPW), dummy], axis=1).reshape(NW, NCH, K)
    xpad = jnp.concatenate(
        [features, jnp.zeros((NP - N, D), jnp.float32)], axis=0)

    degp = _deg_kernel(dst).reshape(NW, NP)
    y0, dinv = _tca(xpad, W0, degp)
    p0 = _seg_kernel(y0, src, dst)
    y1 = _tcb(y0, p0, dinv, b0.reshape(1, D), W1)
    p1 = _seg_kernel(y1, src, dst)
    y2 = _tcb(y1, p1, dinv, b1.reshape(1, D), W2)
    p2 = _seg_kernel(y2, src, dst)
    return _tcc(y2, p2, dinv, b2.reshape(1, D))
